# Initial kernel scaffold; baseline (speedup 1.0000x reference)
import functools

import jax
import jax.numpy as jnp
from jax import lax
from jax.experimental import pallas as pl
from jax.experimental.pallas import tpu as pltpu
from jax.experimental.pallas import tpu_sc as plsc

N = 10000
M = 10000
F = 128
E = 320000

NC = 2
NS = 16
NW = NC * NS
L = 16

NP = 10240
EW = E // NW
K = 79
EWP = K * 128
TS = NP // NS

_BIG_NEG = -1e30


def _mm_scal_body(x_ref, w_ref, a_ref, h_ref, st_ref):
    xb = x_ref[...]
    hb = jnp.dot(xb, w_ref[...], preferred_element_type=jnp.float32)
    h_ref[...] = hb
    st_ref[...] = lax.dot_general(a_ref[...], hb, (((0,), (1,)), ((), ())),
                                  preferred_element_type=jnp.float32)


def _matmul_scal(x, w, acols):
    blk = 1024
    return pl.pallas_call(
        _mm_scal_body,
        grid=(NP // blk,),
        in_specs=[
            pl.BlockSpec((blk, F), lambda i: (i, 0)),
            pl.BlockSpec((F, F), lambda i: (0, 0)),
            pl.BlockSpec((F, 8), lambda i: (0, 0)),
        ],
        out_specs=[
            pl.BlockSpec((blk, F), lambda i: (i, 0)),
            pl.BlockSpec((8, blk), lambda i: (0, i)),
        ],
        out_shape=[
            jax.ShapeDtypeStruct((NP, F), jnp.float32),
            jax.ShapeDtypeStruct((8, NP), jnp.float32),
        ],
    )(x, w, acols)


def _relu_mm_scal_body(f0_ref, f1_ref, w_ref, a_ref, u_ref, st_ref):
    fb = jnp.maximum(f0_ref[0] + f1_ref[0], 0.0)
    ub = jnp.dot(fb, w_ref[...], preferred_element_type=jnp.float32)
    u_ref[...] = ub
    st_ref[...] = lax.dot_general(a_ref[...], ub, (((0,), (1,)), ((), ())),
                                  preferred_element_type=jnp.float32)


def _relu_matmul_scal(fpart, w, acols):
    blk = 1024
    return pl.pallas_call(
        _relu_mm_scal_body,
        grid=(NP // blk,),
        in_specs=[
            pl.BlockSpec((1, blk, F), lambda i: (0, i, 0)),
            pl.BlockSpec((1, blk, F), lambda i: (1, i, 0)),
            pl.BlockSpec((F, F), lambda i: (0, 0)),
            pl.BlockSpec((F, 8), lambda i: (0, 0)),
        ],
        out_specs=[
            pl.BlockSpec((blk, F), lambda i: (i, 0)),
            pl.BlockSpec((8, blk), lambda i: (0, i)),
        ],
        out_shape=[
            jax.ShapeDtypeStruct((NP, F), jnp.float32),
            jax.ShapeDtypeStruct((8, NP), jnp.float32),
        ],
    )(fpart, fpart, w, acols)


def _elu_sum_body(o0_ref, o1_ref, out_ref):
    o = o0_ref[0] + o1_ref[0]
    out_ref[...] = jnp.where(o > 0, o, jnp.expm1(o))


def _elu_sum(opart):
    blk = 1000
    return pl.pallas_call(
        _elu_sum_body,
        grid=(N // blk,),
        in_specs=[
            pl.BlockSpec((1, blk, F), lambda i: (0, i, 0)),
            pl.BlockSpec((1, blk, F), lambda i: (1, i, 0)),
        ],
        out_specs=pl.BlockSpec((blk, F), lambda i: (i, 0)),
        out_shape=jax.ShapeDtypeStruct((N, F), jnp.float32),
    )(opart, opart)


_MESH = plsc.VectorSubcoreMesh(core_axis_name="c", subcore_axis_name="s",
                               num_cores=NC, num_subcores=NS)


def _scalar_pass_body(two_tables, tab_row, tab2_row,
                      tab_hbm, tab2_hbm, gidx_hbm, g2idx_hbm, sidx_hbm,
                      e_hbm, den_hbm,
                      tab_v, tab2_v, gidx_v, g2idx_v, sidx_v, ev, zbuf,
                      den_sh):
    c = lax.axis_index("c")
    s = lax.axis_index("s")
    wid = s * NC + c

    pltpu.sync_copy(tab_hbm.at[tab_row], tab_v)
    pltpu.sync_copy(gidx_hbm.at[wid], gidx_v)
    pltpu.sync_copy(sidx_hbm.at[wid], sidx_v)
    if two_tables:
        pltpu.sync_copy(tab2_hbm.at[tab2_row], tab2_v)
        pltpu.sync_copy(g2idx_hbm.at[wid], g2idx_v)

    for t in range((NP - N) // L):
        tab_v[pl.ds(N + L * t, L)] = jnp.full((L,), _BIG_NEG, jnp.float32)

    for i in range(TS // L):
        zbuf[pl.ds(L * i, L)] = jnp.zeros((L,), jnp.float32)
    pltpu.sync_copy(zbuf, den_sh.at[pl.ds(TS * s, TS)])
    plsc.subcore_barrier()

    def compute(j, _):
        for cc in range(8):
            sl = pl.ds(L * cc, L)
            v = plsc.load_gather(tab_v, [gidx_v[j, sl]])
            if two_tables:
                v = v + plsc.load_gather(tab2_v, [g2idx_v[j, sl]])
            v = jnp.where(v > 0, v, 0.2 * v)
            ev[j, sl] = jnp.exp(v)
        return 0

    lax.fori_loop(0, K, compute, 0)

    def scatter(j, _):
        pltpu.sync_copy(ev.at[j], den_sh.at[sidx_v.at[j]], add=True)
        return 0

    lax.fori_loop(0, K, scatter, 0)
    plsc.subcore_barrier()

    pltpu.sync_copy(ev, e_hbm.at[wid])
    sl = pl.ds(TS * s, TS)
    pltpu.sync_copy(den_sh.at[sl], den_hbm.at[c, sl])


def _make_scalar_pass(two_tables, tab_row, tab2_row):
    body = functools.partial(_scalar_pass_body, two_tables, tab_row, tab2_row)
    return pl.kernel(
        body,
        out_type=[
            jax.ShapeDtypeStruct((NW, K, 128), jnp.float32),
            jax.ShapeDtypeStruct((NC, NP), jnp.float32),
        ],
        mesh=_MESH,
        scratch_types=[
            pltpu.VMEM((NP,), jnp.float32),
            pltpu.VMEM((NP,), jnp.float32),
            pltpu.VMEM((K, 128), jnp.int32),
            pltpu.VMEM((K, 128), jnp.int32),
            pltpu.VMEM((K, 128), jnp.int32),
            pltpu.VMEM((K, 128), jnp.float32),
            pltpu.VMEM((TS,), jnp.float32),
            pltpu.VMEM_SHARED((NP,), jnp.float32),
        ],
    )


def _row_pass_body(rows_hbm, e_hbm, den_hbm, gidx_hbm, sidx_hbm, acc_hbm,
                   den_v, den2_v, gidx_v, sidx_v, ev, wbuf, rows_v, acc_sh):
    c = lax.axis_index("c")
    s = lax.axis_index("s")
    wid = s * NC + c

    pltpu.sync_copy(den_hbm.at[0], den_v)
    pltpu.sync_copy(den_hbm.at[1], den2_v)
    pltpu.sync_copy(gidx_hbm.at[wid], gidx_v)
    pltpu.sync_copy(sidx_hbm.at[wid], sidx_v)
    pltpu.sync_copy(e_hbm.at[wid], ev)

    def den_combine(i, _):
        sl = pl.ds(L * i, L)
        den_v[sl] = den_v[sl] + (den2_v[sl] + 1e-16)
        return 0

    lax.fori_loop(0, NP // L, den_combine, 0)

    def zrow(r, _):
        for cc in range(8):
            rows_v[r, pl.ds(L * cc, L)] = jnp.zeros((L,), jnp.float32)
        return 0

    lax.fori_loop(0, 128, zrow, 0)
    for b in range(TS // 128):
        pltpu.sync_copy(rows_v, acc_sh.at[pl.ds(TS * s + 128 * b, 128)])
    plsc.subcore_barrier()

    def chunk(j, _):
        pltpu.sync_copy(rows_hbm.at[gidx_v.at[j]], rows_v)
        for cc in range(8):
            sl = pl.ds(L * cc, L)
            d = plsc.load_gather(den_v, [sidx_v[j, sl]])
            wbuf[sl] = ev[j, sl] / d

        def scale(r, _):
            w = plsc.load_gather(wbuf, [jnp.full((L,), r, jnp.int32)])
            for cc in range(8):
                sl = pl.ds(L * cc, L)
                rows_v[r, sl] = rows_v[r, sl] * w
            return 0

        lax.fori_loop(0, 128, scale, 0)
        pltpu.sync_copy(rows_v, acc_sh.at[sidx_v.at[j]], add=True)
        return 0

    lax.fori_loop(0, K, chunk, 0)
    plsc.subcore_barrier()

    sl = pl.ds(TS * s, TS)
    pltpu.sync_copy(acc_sh.at[sl], acc_hbm.at[c, sl])


_row_pass = pl.kernel(
    _row_pass_body,
    out_type=[jax.ShapeDtypeStruct((NC, NP, F), jnp.float32)],
    mesh=_MESH,
    scratch_types=[
        pltpu.VMEM((NP,), jnp.float32),
        pltpu.VMEM((NP,), jnp.float32),
        pltpu.VMEM((K, 128), jnp.int32),
        pltpu.VMEM((K, 128), jnp.int32),
        pltpu.VMEM((K, 128), jnp.float32),
        pltpu.VMEM((128,), jnp.float32),
        pltpu.VMEM((128, F), jnp.float32),
        pltpu.VMEM_SHARED((NP, F), jnp.float32),
    ],
)


def _pad_idx(a, fill):
    a = a.reshape(NW, EW)
    a = jnp.pad(a, ((0, 0), (0, EWP - EW)), constant_values=fill)
    return a.reshape(NW, K, 128)


def kernel(x, edge_index, W1, a1, W2, a2):
    src = edge_index[0].astype(jnp.int32)
    eidx = edge_index[1].astype(jnp.int32)
    src3 = _pad_idx(src, N)
    eidx3 = _pad_idx(eidx, 0)

    x_pad = jnp.pad(x, ((0, NP - N), (0, 0)))
    acols = jnp.zeros((F, 8), jnp.float32)
    acols = acols.at[:, 0].set(a1).at[:, 1].set(a2[:F])
    a2b = jnp.zeros((F, 8), jnp.float32).at[:, 0].set(a2[F:])

    h, scalT = _matmul_scal(x_pad, W1, acols)

    e1, den1 = _make_scalar_pass(False, 0, 0)(scalT, scalT, src3, src3, eidx3)
    (fpart,) = _row_pass(h, e1, den1, src3, eidx3)

    u, uscalT = _relu_matmul_scal(fpart, W2, a2b)

    e2, den2 = _make_scalar_pass(True, 1, 0)(scalT, uscalT, src3, eidx3, src3)
    (opart,) = _row_pass(u, e2, den2, eidx3, src3)

    return _elu_sum(opart)


# speedup vs baseline: 7.6782x; 7.6782x over previous
import functools

import jax
import jax.numpy as jnp
from jax import lax
from jax.experimental import pallas as pl
from jax.experimental.pallas import tpu as pltpu
from jax.experimental.pallas import tpu_sc as plsc

N = 10000
M = 10000
F = 128
E = 320000

NC = 2
NS = 16
NW = NC * NS
L = 16

NP = 10240
HN = NP // NC
EW = E // NW
K = 79
EWP = K * 128
TS = NP // NS
HTS = HN // NS

_BIG_NEG = -1e30


def _mm_scal_body(x_ref, w_ref, a_ref, h_ref, st_ref):
    xb = x_ref[...]
    hb = jnp.dot(xb, w_ref[...], preferred_element_type=jnp.float32)
    h_ref[...] = hb
    st_ref[...] = lax.dot_general(a_ref[...], hb, (((0,), (1,)), ((), ())),
                                  preferred_element_type=jnp.float32)


def _matmul_scal(x, w, acols, relu_input=False):
    blk = 1024
    body = _relu_mm_scal_body if relu_input else _mm_scal_body
    return pl.pallas_call(
        body,
        grid=(NP // blk,),
        in_specs=[
            pl.BlockSpec((blk, F), lambda i: (i, 0)),
            pl.BlockSpec((F, F), lambda i: (0, 0)),
            pl.BlockSpec((F, 8), lambda i: (0, 0)),
        ],
        out_specs=[
            pl.BlockSpec((blk, F), lambda i: (i, 0)),
            pl.BlockSpec((8, blk), lambda i: (0, i)),
        ],
        out_shape=[
            jax.ShapeDtypeStruct((NP, F), jnp.float32),
            jax.ShapeDtypeStruct((8, NP), jnp.float32),
        ],
    )(x, w, acols)


def _relu_mm_scal_body(f_ref, w_ref, a_ref, u_ref, st_ref):
    fb = jnp.maximum(f_ref[...], 0.0)
    ub = jnp.dot(fb, w_ref[...], preferred_element_type=jnp.float32)
    u_ref[...] = ub
    st_ref[...] = lax.dot_general(a_ref[...], ub, (((0,), (1,)), ((), ())),
                                  preferred_element_type=jnp.float32)


def _elu_body(o_ref, out_ref):
    o = o_ref[...]
    out_ref[...] = jnp.where(o > 0, o, jnp.exp(o) - 1.0)


def _elu(o_full):
    blk = 1000
    return pl.pallas_call(
        _elu_body,
        grid=(N // blk,),
        in_specs=[pl.BlockSpec((blk, F), lambda i: (i, 0))],
        out_specs=pl.BlockSpec((blk, F), lambda i: (i, 0)),
        out_shape=jax.ShapeDtypeStruct((N, F), jnp.float32),
    )(o_full)


_MESH = plsc.VectorSubcoreMesh(core_axis_name="c", subcore_axis_name="s",
                               num_cores=NC, num_subcores=NS)


def _scalar_pass_body(two_tables, tab_row, tab2_row,
                      tab_hbm, tab2_hbm, gidx_hbm, g2idx_hbm, sidx_hbm,
                      e_hbm, den_hbm,
                      tab_v, tab2_v, gidx_v, g2idx_v, sidx_v, ev, zbuf,
                      den_sh):
    c = lax.axis_index("c")
    s = lax.axis_index("s")
    wid = s * NC + c

    pltpu.sync_copy(tab_hbm.at[tab_row], tab_v)
    pltpu.sync_copy(gidx_hbm.at[wid], gidx_v)
    pltpu.sync_copy(sidx_hbm.at[wid], sidx_v)
    if two_tables:
        pltpu.sync_copy(tab2_hbm.at[tab2_row], tab2_v)
        pltpu.sync_copy(g2idx_hbm.at[wid], g2idx_v)

    for t in range((NP - N) // L):
        tab_v[pl.ds(N + L * t, L)] = jnp.full((L,), _BIG_NEG, jnp.float32)

    for i in range(TS // L):
        zbuf[pl.ds(L * i, L)] = jnp.zeros((L,), jnp.float32)
    pltpu.sync_copy(zbuf, den_sh.at[pl.ds(TS * s, TS)])
    plsc.subcore_barrier()

    def compute(j, _):
        for cc in range(8):
            sl = pl.ds(L * cc, L)
            v = plsc.load_gather(tab_v, [gidx_v[j, sl]])
            if two_tables:
                v = v + plsc.load_gather(tab2_v, [g2idx_v[j, sl]])
            v = jnp.where(v > 0, v, 0.2 * v)
            ev[j, sl] = jnp.exp(v)
        return 0

    lax.fori_loop(0, K, compute, 0)

    def scatter(j, _):
        pltpu.sync_copy(ev.at[j], den_sh.at[sidx_v.at[j]], add=True)
        return 0

    lax.fori_loop(0, K, scatter, 0)
    plsc.subcore_barrier()

    pltpu.sync_copy(ev, e_hbm.at[wid])
    sl = pl.ds(TS * s, TS)
    pltpu.sync_copy(den_sh.at[sl], den_hbm.at[c, sl])


def _make_scalar_pass(two_tables, tab_row, tab2_row):
    body = functools.partial(_scalar_pass_body, two_tables, tab_row, tab2_row)
    return pl.kernel(
        body,
        out_type=[
            jax.ShapeDtypeStruct((NW, K, 128), jnp.float32),
            jax.ShapeDtypeStruct((NC, NP), jnp.float32),
        ],
        mesh=_MESH,
        compiler_params=pltpu.CompilerParams(needs_layout_passes=False),
        scratch_types=[
            pltpu.VMEM((NP,), jnp.float32),
            pltpu.VMEM((NP,), jnp.float32),
            pltpu.VMEM((K, 128), jnp.int32),
            pltpu.VMEM((K, 128), jnp.int32),
            pltpu.VMEM((K, 128), jnp.int32),
            pltpu.VMEM((K, 128), jnp.float32),
            pltpu.VMEM((TS,), jnp.float32),
            pltpu.VMEM_SHARED((NP,), jnp.float32),
        ],
    )


def _row_pass_body(rows_hbm, e_hbm, den_hbm, gidx_hbm, sidx_hbm, acc_hbm,
                   den_v, den2_v, gidx_v, sidx_v, ev, wbuf, lsx, rows_v,
                   acc_sh):
    c = lax.axis_index("c")
    s = lax.axis_index("s")
    base = c * HN

    pltpu.sync_copy(den_hbm.at[0], den_v)
    pltpu.sync_copy(den_hbm.at[1], den2_v)

    def den_combine(i, _):
        sl = pl.ds(L * i, L)
        den_v[sl] = den_v[sl] + (den2_v[sl] + 1e-16)
        return 0

    lax.fori_loop(0, NP // L, den_combine, 0)

    def zrow(r, _):
        for cc in range(8):
            rows_v[r, pl.ds(L * cc, L)] = jnp.zeros((L,), jnp.float32)
        return 0

    lax.fori_loop(0, 128, zrow, 0)
    for b in range(HTS // 128):
        pltpu.sync_copy(rows_v, acc_sh.at[pl.ds(HTS * s + 128 * b, 128)])
    pltpu.sync_copy(rows_v.at[pl.ds(0, HTS % 128)],
                    acc_sh.at[pl.ds(HTS * s + (HTS // 128) * 128, HTS % 128)])
    plsc.subcore_barrier()

    for wsub in range(NC):
        wid = s * NC + wsub
        pltpu.sync_copy(gidx_hbm.at[wid], gidx_v)
        pltpu.sync_copy(sidx_hbm.at[wid], sidx_v)
        pltpu.sync_copy(e_hbm.at[wid], ev)

        def chunk(j, _):
            pltpu.sync_copy(rows_hbm.at[gidx_v.at[j]], rows_v)
            for cc in range(8):
                sl = pl.ds(L * cc, L)
                sidx16 = sidx_v[j, sl]
                d = plsc.load_gather(den_v, [sidx16])
                lidx = sidx16 - base
                msk = (lidx >= 0) & (lidx < HN)
                lsx[0, sl] = jnp.where(msk, lidx, 0)
                wbuf[sl] = jnp.where(msk, ev[j, sl] / d, 0.0)

            def scale(r, _):
                w = plsc.load_gather(wbuf, [jnp.full((L,), r, jnp.int32)])
                for cc in range(8):
                    sl = pl.ds(L * cc, L)
                    rows_v[r, sl] = rows_v[r, sl] * w
                return 0

            lax.fori_loop(0, 128, scale, 0)
            pltpu.sync_copy(rows_v, acc_sh.at[lsx.at[0]], add=True)
            return 0

        lax.fori_loop(0, K, chunk, 0)

    plsc.subcore_barrier()
    sl = pl.ds(HTS * s, HTS)
    pltpu.sync_copy(acc_sh.at[sl], acc_hbm.at[c, sl])


_row_pass = pl.kernel(
    _row_pass_body,
    out_type=[jax.ShapeDtypeStruct((NC, HN, F), jnp.float32)],
    mesh=_MESH,
    compiler_params=pltpu.CompilerParams(needs_layout_passes=False),
    scratch_types=[
        pltpu.VMEM((NP,), jnp.float32),
        pltpu.VMEM((NP,), jnp.float32),
        pltpu.VMEM((K, 128), jnp.int32),
        pltpu.VMEM((K, 128), jnp.int32),
        pltpu.VMEM((K, 128), jnp.float32),
        pltpu.VMEM((128,), jnp.float32),
        pltpu.VMEM((1, 128), jnp.int32),
        pltpu.VMEM((128, F), jnp.float32),
        pltpu.VMEM_SHARED((HN, F), jnp.float32),
    ],
)


def _pad_idx(a, fill):
    a = a.reshape(NW, EW)
    a = jnp.pad(a, ((0, 0), (0, EWP - EW)), constant_values=fill)
    return a.reshape(NW, K, 128)


def kernel(x, edge_index, W1, a1, W2, a2):
    src = edge_index[0].astype(jnp.int32)
    eidx = edge_index[1].astype(jnp.int32)
    src3 = _pad_idx(src, N)
    eidx3 = _pad_idx(eidx, 0)

    x_pad = jnp.pad(x, ((0, NP - N), (0, 0)))
    acols = jnp.zeros((F, 8), jnp.float32)
    acols = acols.at[:, 0].set(a1).at[:, 1].set(a2[:F])
    a2b = jnp.zeros((F, 8), jnp.float32).at[:, 0].set(a2[F:])

    h, scalT = _matmul_scal(x_pad, W1, acols)

    e1, den1 = _make_scalar_pass(False, 0, 0)(scalT, scalT, src3, src3, eidx3)
    (fpart,) = _row_pass(h, e1, den1, src3, eidx3)
    f_full = fpart.reshape(NP, F)

    u, uscalT = _matmul_scal(f_full, W2, a2b, relu_input=True)

    e2, den2 = _make_scalar_pass(True, 1, 0)(scalT, uscalT, src3, eidx3, src3)
    (opart,) = _row_pass(u, e2, den2, eidx3, src3)

    return _elu(opart.reshape(NP, F))


# speedup vs baseline: 7.9426x; 1.0344x over previous
import functools

import jax
import jax.numpy as jnp
from jax import lax
from jax.experimental import pallas as pl
from jax.experimental.pallas import tpu as pltpu
from jax.experimental.pallas import tpu_sc as plsc

N = 10000
M = 10000
F = 128
E = 320000

NC = 2
NS = 16
NW = NC * NS
L = 16

NP = 10240
HN = NP // NC
EW = E // NW
K = 80
EWP = K * 128
TS = NP // NS
HTS = HN // NS

_BIG_NEG = -1e30


def _mm_scal_body(x_ref, w_ref, a_ref, h_ref, st_ref):
    xb = x_ref[...]
    hb = jnp.dot(xb, w_ref[...], preferred_element_type=jnp.float32)
    h_ref[...] = hb
    st_ref[...] = lax.dot_general(a_ref[...], hb, (((0,), (1,)), ((), ())),
                                  preferred_element_type=jnp.float32)


def _matmul_scal(x, w, acols, relu_input=False):
    blk = 1024
    body = _relu_mm_scal_body if relu_input else _mm_scal_body
    return pl.pallas_call(
        body,
        grid=(NP // blk,),
        in_specs=[
            pl.BlockSpec((blk, F), lambda i: (i, 0)),
            pl.BlockSpec((F, F), lambda i: (0, 0)),
            pl.BlockSpec((F, 8), lambda i: (0, 0)),
        ],
        out_specs=[
            pl.BlockSpec((blk, F), lambda i: (i, 0)),
            pl.BlockSpec((8, blk), lambda i: (0, i)),
        ],
        out_shape=[
            jax.ShapeDtypeStruct((NP, F), jnp.float32),
            jax.ShapeDtypeStruct((8, NP), jnp.float32),
        ],
    )(x, w, acols)


def _relu_mm_scal_body(f_ref, w_ref, a_ref, u_ref, st_ref):
    fb = jnp.maximum(f_ref[...], 0.0)
    ub = jnp.dot(fb, w_ref[...], preferred_element_type=jnp.float32)
    u_ref[...] = ub
    st_ref[...] = lax.dot_general(a_ref[...], ub, (((0,), (1,)), ((), ())),
                                  preferred_element_type=jnp.float32)


def _elu_body(o_ref, out_ref):
    o = o_ref[...]
    out_ref[...] = jnp.where(o > 0, o, jnp.exp(o) - 1.0)


def _elu(o_full):
    blk = 1000
    return pl.pallas_call(
        _elu_body,
        grid=(N // blk,),
        in_specs=[pl.BlockSpec((blk, F), lambda i: (i, 0))],
        out_specs=pl.BlockSpec((blk, F), lambda i: (i, 0)),
        out_shape=jax.ShapeDtypeStruct((N, F), jnp.float32),
    )(o_full)


_MESH = plsc.VectorSubcoreMesh(core_axis_name="c", subcore_axis_name="s",
                               num_cores=NC, num_subcores=NS)


CH = 64
PCAP = K * 128 + 4 * CH


def _row_pass_body(two_tables, t1_row, t2_row,
                   rows_hbm, t1_hbm, t2_hbm, gidx_hbm, sidx_hbm, acc_hbm,
                   t1_v, t2_v, gidx_v, sidx_v, pend_g, pend_l, pend_e,
                   lsx, evrow, rd_v, rows_a, rows_b, acc_sh, den_sh,
                   gsem_a, gsem_b):
    c = lax.axis_index("c")
    s = lax.axis_index("s")
    base = c * HN

    pltpu.sync_copy(t1_hbm.at[t1_row], t1_v)
    if two_tables:
        pltpu.sync_copy(t2_hbm.at[t2_row], t2_v)

    for t in range((NP - N) // L):
        t1_v[pl.ds(N + L * t, L)] = jnp.full((L,), _BIG_NEG, jnp.float32)

    def zrow(r, _):
        for cc in range(8):
            rows_a[r, pl.ds(L * cc, L)] = jnp.zeros((L,), jnp.float32)
        return 0

    lax.fori_loop(0, CH, zrow, 0)
    for b in range(HTS // CH):
        pltpu.sync_copy(rows_a, acc_sh.at[pl.ds(HTS * s + CH * b, CH)])
    for v in range(HTS // L):
        rd_v[pl.ds(L * v, L)] = jnp.zeros((L,), jnp.float32)
    pltpu.sync_copy(rd_v, den_sh.at[pl.ds(HTS * s, HTS)])
    plsc.subcore_barrier()

    def fire(j, buf, sem):
        pltpu.async_copy(rows_hbm.at[pend_g.at[pl.ds(CH * j, CH)]],
                         buf, sem)

    def wait(j, buf, sem):
        pltpu.make_async_copy(rows_hbm.at[pend_g.at[pl.ds(CH * j, CH)]],
                              buf, sem).wait()

    def process(j, buf):
        for cc in range(CH // L):
            sl = pl.ds(L * cc, L)
            p = pl.ds(CH * j + L * cc, L)
            lsx[0, sl] = pend_l[p]
            evrow[0, sl] = pend_e[p]

        def scale(r, _):
            w = plsc.load_gather(pend_e, [jnp.full((L,), CH * j + r,
                                                   jnp.int32)])
            for cc in range(8):
                sl = pl.ds(L * cc, L)
                buf[r, sl] = buf[r, sl] * w
            return 0

        lax.fori_loop(0, CH, scale, 0)
        pltpu.sync_copy(buf, acc_sh.at[lsx.at[0]], add=True)
        pltpu.sync_copy(evrow.at[0], den_sh.at[lsx.at[0]], add=True)

    for wsub in range(NC):
        wid = s * NC + wsub
        pltpu.sync_copy(gidx_hbm.at[wid], gidx_v)
        pltpu.sync_copy(sidx_hbm.at[wid], sidx_v)

        def compact(j, cnt):
            for cc in range(8):
                sl = pl.ds(L * cc, L)
                g16 = gidx_v[j, sl]
                s16 = sidx_v[j, sl]
                if two_tables:
                    raw = (plsc.load_gather(t1_v, [s16])
                           + plsc.load_gather(t2_v, [g16]))
                else:
                    raw = plsc.load_gather(t1_v, [g16])
                raw = jnp.where(raw > 0, raw, 0.2 * raw)
                e16 = jnp.exp(raw)
                l16 = s16 - base
                msk = (l16 >= 0) & (l16 < HN)
                mi = msk.astype(jnp.int32)
                pos = cnt + plsc.cumsum(mi) - 1
                plsc.store_scatter(pend_g, [pos], g16, mask=msk)
                plsc.store_scatter(pend_l, [pos], l16, mask=msk)
                plsc.store_scatter(pend_e, [pos], e16, mask=msk)
                cnt = cnt + jnp.sum(mi)
            return cnt

        cnt = lax.fori_loop(0, K, compact, jnp.int32(0))

        iota16 = lax.iota(jnp.int32, L)
        for t in range(4 * CH // L):
            pos = cnt + iota16 + L * t
            plsc.store_scatter(pend_g, [pos], jnp.zeros((L,), jnp.int32))
            plsc.store_scatter(pend_l, [pos], jnp.zeros((L,), jnp.int32))
            plsc.store_scatter(pend_e, [pos], jnp.zeros((L,), jnp.float32))

        npair = ((cnt + CH - 1) // CH + 1) // 2

        fire(0, rows_a, gsem_a)
        fire(1, rows_b, gsem_b)

        def pair(jj, _):
            j0 = 2 * jj
            wait(j0, rows_a, gsem_a)
            process(j0, rows_a)
            fire(j0 + 2, rows_a, gsem_a)
            wait(j0 + 1, rows_b, gsem_b)
            process(j0 + 1, rows_b)
            fire(j0 + 3, rows_b, gsem_b)
            return 0

        lax.fori_loop(0, npair, pair, 0)
        wait(2 * npair, rows_a, gsem_a)
        wait(2 * npair + 1, rows_b, gsem_b)

    plsc.subcore_barrier()

    pltpu.sync_copy(den_sh.at[pl.ds(HTS * s, HTS)], rd_v)
    for v in range(HTS // L):
        sl = pl.ds(L * v, L)
        rd_v[sl] = 1.0 / (rd_v[sl] + 1e-16)
    for b in range(HTS // 64):
        r0 = HTS * s + 64 * b
        pltpu.sync_copy(acc_sh.at[pl.ds(r0, 64)], rows_a)

        def nrow(r, _):
            w = plsc.load_gather(rd_v, [jnp.full((L,), 64 * b + r,
                                                 jnp.int32)])
            for cc in range(8):
                sl = pl.ds(L * cc, L)
                rows_a[r, sl] = rows_a[r, sl] * w
            return 0

        lax.fori_loop(0, 64, nrow, 0)
        pltpu.sync_copy(rows_a, acc_hbm.at[c, pl.ds(r0, 64)])


def _make_row_pass(two_tables, t1_row, t2_row):
    body = functools.partial(_row_pass_body, two_tables, t1_row, t2_row)
    return pl.kernel(
        body,
        out_type=[jax.ShapeDtypeStruct((NC, HN, F), jnp.float32)],
        mesh=_MESH,
        compiler_params=pltpu.CompilerParams(needs_layout_passes=False),
        scratch_types=[
            pltpu.VMEM((NP,), jnp.float32),
            pltpu.VMEM((NP,), jnp.float32),
            pltpu.VMEM((K, 128), jnp.int32),
            pltpu.VMEM((K, 128), jnp.int32),
            pltpu.VMEM((PCAP,), jnp.int32),
            pltpu.VMEM((PCAP,), jnp.int32),
            pltpu.VMEM((PCAP,), jnp.float32),
            pltpu.VMEM((1, CH), jnp.int32),
            pltpu.VMEM((1, CH), jnp.float32),
            pltpu.VMEM((HTS,), jnp.float32),
            pltpu.VMEM((CH, F), jnp.float32),
            pltpu.VMEM((CH, F), jnp.float32),
            pltpu.VMEM_SHARED((HN, F), jnp.float32),
            pltpu.VMEM_SHARED((HN,), jnp.float32),
            pltpu.SemaphoreType.DMA,
            pltpu.SemaphoreType.DMA,
        ],
    )


def _pad_idx(a, fill):
    a = a.reshape(NW, EW)
    a = jnp.pad(a, ((0, 0), (0, EWP - EW)), constant_values=fill)
    return a.reshape(NW, K, 128)


def kernel(x, edge_index, W1, a1, W2, a2):
    src = edge_index[0].astype(jnp.int32)
    eidx = edge_index[1].astype(jnp.int32)
    src3 = _pad_idx(src, N)
    eidx3 = _pad_idx(eidx, 0)

    x_pad = jnp.pad(x, ((0, NP - N), (0, 0)))
    acols = jnp.zeros((F, 8), jnp.float32)
    acols = acols.at[:, 0].set(a1).at[:, 1].set(a2[:F])
    a2b = jnp.zeros((F, 8), jnp.float32).at[:, 0].set(a2[F:])

    h, scalT = _matmul_scal(x_pad, W1, acols)

    (fpart,) = _make_row_pass(False, 0, 0)(h, scalT, scalT, src3, eidx3)

    u, uscalT = _matmul_scal(fpart.reshape(NP, F), W2, a2b, relu_input=True)

    (opart,) = _make_row_pass(True, 1, 0)(u, scalT, uscalT, eidx3, src3)

    return _elu(opart.reshape(NP, F))
